# baseline (device time: 200941 ns/iter reference)
import jax
import jax.numpy as jnp
from jax import lax
from jax.experimental import pallas as pl
from jax.experimental.pallas import tpu as pltpu

T = 2048
D = 4096
V_SHARD = 8192
BV = 512
NBLK = V_SHARD // BV
CH = 256
NCH = T // CH

_S, _L = 0, 1


def kernel(x, W, labels):
    labels2d = labels.reshape(T, 1)

    def body(x_hbm, w_ref, lbl_ref, out_ref,
             xb, land, stats, rx, copy_sems, send_sem, recv_sem, ack_sem):
        j = pl.program_id(0)
        my_x = lax.axis_index("x")
        my_y = lax.axis_index("y")
        my_z = lax.axis_index("z")

        @pl.when(j == 0)
        def _():
            first = pltpu.make_async_copy(
                x_hbm.at[pl.ds(0, CH), :], land.at[0], copy_sems.at[0])
            first.start()
            for c in range(NCH):
                if c + 1 < NCH:
                    nxt = pltpu.make_async_copy(
                        x_hbm.at[pl.ds((c + 1) * CH, CH), :],
                        land.at[(c + 1) % 2], copy_sems.at[(c + 1) % 2])
                    nxt.start()
                cur = pltpu.make_async_copy(
                    x_hbm.at[pl.ds(c * CH, CH), :],
                    land.at[c % 2], copy_sems.at[c % 2])
                cur.wait()
                xb[pl.ds(c * CH, CH), :] = land[c % 2].astype(jnp.bfloat16)

        w_bf = w_ref[...].astype(jnp.bfloat16)
        logits = jnp.dot(xb[...], w_bf,
                         preferred_element_type=jnp.float32)

        offset = my_x * V_SHARD + j * BV
        idx = lbl_ref[...] - offset
        col = lax.broadcasted_iota(jnp.int32, (T, BV), 1)
        lval = jnp.sum(jnp.where(col == idx, logits, 0.0),
                       axis=1, keepdims=True)
        s = jnp.sum(jnp.exp(logits), axis=1, keepdims=True)

        @pl.when(j == 0)
        def _():
            stats[:, _S:_S + 1] = s
            stats[:, _L:_L + 1] = lval

        @pl.when(j > 0)
        def _():
            stats[:, _S:_S + 1] = stats[:, _S:_S + 1] + s
            stats[:, _L:_L + 1] = stats[:, _L:_L + 1] + lval

        @pl.when(j == NBLK - 1)
        def _():
            partner = (1 - my_x, my_y, my_z)
            rdma = pltpu.make_async_remote_copy(
                src_ref=stats, dst_ref=rx,
                send_sem=send_sem, recv_sem=recv_sem,
                device_id=partner,
                device_id_type=pl.DeviceIdType.MESH)
            rdma.start()
            rdma.wait()

            s_tot = stats[:, _S:_S + 1] + rx[:, _S:_S + 1]
            l_tot = stats[:, _L:_L + 1] + rx[:, _L:_L + 1]
            out_ref[...] = jnp.log(s_tot) - l_tot

            pl.semaphore_signal(ack_sem, 1, device_id=partner,
                                device_id_type=pl.DeviceIdType.MESH)
            pl.semaphore_wait(ack_sem, 1)

    out = pl.pallas_call(
        body,
        grid=(NBLK,),
        in_specs=[
            pl.BlockSpec(memory_space=pltpu.MemorySpace.HBM),
            pl.BlockSpec((D, BV), lambda j: (0, j)),
            pl.BlockSpec((T, 1), lambda j: (0, 0)),
        ],
        out_specs=pl.BlockSpec((T, 1), lambda j: (0, 0)),
        out_shape=jax.ShapeDtypeStruct((T, 1), jnp.float32),
        scratch_shapes=[
            pltpu.VMEM((T, D), jnp.bfloat16),
            pltpu.VMEM((2, CH, D), jnp.float32),
            pltpu.VMEM((T, 128), jnp.float32),
            pltpu.VMEM((T, 128), jnp.float32),
            pltpu.SemaphoreType.DMA((2,)),
            pltpu.SemaphoreType.DMA,
            pltpu.SemaphoreType.DMA,
            pltpu.SemaphoreType.REGULAR,
        ],
        compiler_params=pltpu.CompilerParams(
            dimension_semantics=("arbitrary",),
            vmem_limit_bytes=64 * 1024 * 1024,
        ),
    )(x, W, labels2d)
    return out.reshape(T)
